# baseline (speedup 1.0000x reference)
import functools

import jax
import jax.numpy as jnp
from jax.experimental import pallas as pl
from jax.experimental.pallas import tpu as pltpu

B = 256
DV = 1024
DT = 768
DP = 384
E = 8
TOPK = 2
DFF = 2048
NC = 1000
TEMP = 0.1
DFF_BLK = 512
NK = DFF // DFF_BLK

_SQRT_HALF = 0.7071067811865476


def _gelu(x):
    return x * 0.5 * (1.0 + jax.lax.erf(x * _SQRT_HALF))


def _router_kernel(text_ref, Wt_ref, Wm_ref, bm_ref, Wip_ref, bip_ref, noise_ref,
                   tproj_ref, mask_ref, il_ref, ent_ref):
    tf = jnp.dot(text_ref[...], Wt_ref[...], preferred_element_type=jnp.float32)
    logits = (jnp.dot(tf, Wm_ref[...], preferred_element_type=jnp.float32)
              + bm_ref[...]) / TEMP + noise_ref[...]
    mx = jnp.max(logits, axis=1, keepdims=True)
    ex = jnp.exp(logits - mx)
    scores = ex / jnp.sum(ex, axis=1, keepdims=True)
    iota = jax.lax.broadcasted_iota(jnp.int32, (B, E), 1)
    m1 = jnp.max(scores, axis=1, keepdims=True)
    i1 = jnp.min(jnp.where(scores == m1, iota, E), axis=1, keepdims=True)
    s2 = jnp.where(iota == i1, -jnp.inf, scores)
    m2 = jnp.max(s2, axis=1, keepdims=True)
    i2 = jnp.min(jnp.where(s2 == m2, iota, E), axis=1, keepdims=True)
    mask_ref[...] = ((iota == i1) | (iota == i2)).astype(jnp.float32)
    sum_scores = jnp.sum(scores, axis=0)
    mean_s = jnp.mean(sum_scores)
    var = jnp.sum((sum_scores - mean_s) ** 2) / (E - 1)
    il = var / (mean_s * mean_s)
    il_ref[0, 0] = jnp.where(il > 0.05, il, 0.0)
    ent_ref[0, 0] = jnp.mean(-jnp.sum(scores * jnp.log(scores + 1e-7), axis=1))
    tp = jnp.dot(tf, Wip_ref[...], preferred_element_type=jnp.float32) + bip_ref[...]
    tproj_ref[...] = _gelu(tp)


def _expert_kernel(vis_ref, tproj_ref, mask_ref, W1v_ref, W1t_ref, b1_ref,
                   W2_ref, b2_ref, out_ref):
    e = pl.program_id(0)
    k = pl.program_id(1)

    @pl.when((e == 0) & (k == 0))
    def _init():
        out_ref[...] = jnp.zeros_like(out_ref)

    bf = jnp.bfloat16
    h = _gelu(
        jnp.dot(vis_ref[...].astype(bf), W1v_ref[0].astype(bf),
                preferred_element_type=jnp.float32)
        + jnp.dot(tproj_ref[...].astype(bf), W1t_ref[0].astype(bf),
                  preferred_element_type=jnp.float32)
        + b1_ref[0])
    part = jnp.dot(h.astype(bf), W2_ref[0].astype(bf),
                   preferred_element_type=jnp.float32)
    mask_col = mask_ref[0]
    acc = mask_col * part

    @pl.when(k == 0)
    def _bias():
        out_ref[...] += mask_col * b2_ref[0]

    out_ref[...] += acc


def kernel(vision_input, text_input, W_text, W_moe, b_moe, W_ip, b_ip,
           W1v, W1t, b1, W2, b2):
    noise = jax.random.normal(jax.random.key(42), (B, E), dtype=jnp.float32) / (E ** 2)

    tproj, mask, il, ent = pl.pallas_call(
        _router_kernel,
        out_shape=[
            jax.ShapeDtypeStruct((B, DP), jnp.float32),
            jax.ShapeDtypeStruct((B, E), jnp.float32),
            jax.ShapeDtypeStruct((1, 1), jnp.float32),
            jax.ShapeDtypeStruct((1, 1), jnp.float32),
        ],
        out_specs=[
            pl.BlockSpec((B, DP), lambda: (0, 0)),
            pl.BlockSpec((B, E), lambda: (0, 0)),
            pl.BlockSpec(memory_space=pltpu.SMEM),
            pl.BlockSpec(memory_space=pltpu.SMEM),
        ],
    )(text_input, W_text, W_moe, b_moe.reshape(1, E), W_ip,
      b_ip.reshape(1, DP), noise)

    logits = pl.pallas_call(
        _expert_kernel,
        grid=(E, NK),
        in_specs=[
            pl.BlockSpec((B, DV), lambda e, k: (0, 0)),
            pl.BlockSpec((B, DP), lambda e, k: (0, 0)),
            pl.BlockSpec((1, B, 1), lambda e, k: (e, 0, 0)),
            pl.BlockSpec((1, DV, DFF_BLK), lambda e, k: (e, 0, k)),
            pl.BlockSpec((1, DP, DFF_BLK), lambda e, k: (e, 0, k)),
            pl.BlockSpec((1, 1, DFF_BLK), lambda e, k: (e, 0, k)),
            pl.BlockSpec((1, DFF_BLK, NC), lambda e, k: (e, k, 0)),
            pl.BlockSpec((1, 1, NC), lambda e, k: (e, 0, 0)),
        ],
        out_specs=pl.BlockSpec((B, NC), lambda e, k: (0, 0)),
        out_shape=jax.ShapeDtypeStruct((B, NC), jnp.float32),
    )(vision_input, tproj, mask.T.reshape(E, B, 1), W1v, W1t,
      b1.reshape(E, 1, DFF), W2, b2.reshape(E, 1, NC))

    return (logits, il.reshape(()), ent.reshape(()))


# speedup vs baseline: 1.0550x; 1.0550x over previous
import jax
import jax.numpy as jnp
from jax.experimental import pallas as pl
from jax.experimental.pallas import tpu as pltpu

B = 256
DV = 1024
DT = 768
DP = 384
E = 8
TOPK = 2
DFF = 2048
NC = 1000
TEMP = 0.1

_SQRT_HALF = 0.7071067811865476


def _gelu(x):
    return x * 0.5 * (1.0 + jax.lax.erf(x * _SQRT_HALF))


def _router_kernel(text_ref, Wt_ref, Wm_ref, bm_ref, Wip_ref, bip_ref, noise_ref,
                   tproj_ref, mask_ref, il_ref, ent_ref):
    tf = jnp.dot(text_ref[...], Wt_ref[...], preferred_element_type=jnp.float32)
    logits = (jnp.dot(tf, Wm_ref[...], preferred_element_type=jnp.float32)
              + bm_ref[...]) / TEMP + noise_ref[...]
    mx = jnp.max(logits, axis=1, keepdims=True)
    ex = jnp.exp(logits - mx)
    scores = ex / jnp.sum(ex, axis=1, keepdims=True)
    iota = jax.lax.broadcasted_iota(jnp.int32, (B, E), 1)
    m1 = jnp.max(scores, axis=1, keepdims=True)
    i1 = jnp.min(jnp.where(scores == m1, iota, E), axis=1, keepdims=True)
    s2 = jnp.where(iota == i1, -jnp.inf, scores)
    m2 = jnp.max(s2, axis=1, keepdims=True)
    i2 = jnp.min(jnp.where(s2 == m2, iota, E), axis=1, keepdims=True)
    mask_ref[...] = ((iota == i1) | (iota == i2)).astype(jnp.float32)
    sum_scores = jnp.sum(scores, axis=0)
    mean_s = jnp.mean(sum_scores)
    var = jnp.sum((sum_scores - mean_s) ** 2) / (E - 1)
    il = var / (mean_s * mean_s)
    il_ref[0, 0] = jnp.where(il > 0.05, il, 0.0)
    ent_ref[0, 0] = jnp.mean(-jnp.sum(scores * jnp.log(scores + 1e-7), axis=1))
    tp = jnp.dot(tf, Wip_ref[...], preferred_element_type=jnp.float32) + bip_ref[...]
    tproj_ref[...] = _gelu(tp)


def _expert_kernel(vis_ref, tproj_ref, mask_ref, W1v_ref, W1t_ref, b1_ref,
                   W2_ref, b2_ref, out_ref):
    e = pl.program_id(0)

    @pl.when(e == 0)
    def _init():
        out_ref[...] = jnp.zeros_like(out_ref)

    bf = jnp.bfloat16
    h = _gelu(
        jnp.dot(vis_ref[...].astype(bf), W1v_ref[0].astype(bf),
                preferred_element_type=jnp.float32)
        + jnp.dot(tproj_ref[...].astype(bf), W1t_ref[0].astype(bf),
                  preferred_element_type=jnp.float32)
        + b1_ref[0])
    part = jnp.dot(h.astype(bf), W2_ref[0].astype(bf),
                   preferred_element_type=jnp.float32)
    mask_col = mask_ref[0]
    out_ref[...] += mask_col * (part + b2_ref[0])


def kernel(vision_input, text_input, W_text, W_moe, b_moe, W_ip, b_ip,
           W1v, W1t, b1, W2, b2):
    noise = jax.random.normal(jax.random.key(42), (B, E), dtype=jnp.float32) / (E ** 2)

    tproj, mask, il, ent = pl.pallas_call(
        _router_kernel,
        out_shape=[
            jax.ShapeDtypeStruct((B, DP), jnp.float32),
            jax.ShapeDtypeStruct((B, E), jnp.float32),
            jax.ShapeDtypeStruct((1, 1), jnp.float32),
            jax.ShapeDtypeStruct((1, 1), jnp.float32),
        ],
        out_specs=[
            pl.BlockSpec((B, DP), lambda: (0, 0)),
            pl.BlockSpec((B, E), lambda: (0, 0)),
            pl.BlockSpec(memory_space=pltpu.SMEM),
            pl.BlockSpec(memory_space=pltpu.SMEM),
        ],
    )(text_input, W_text, W_moe, b_moe.reshape(1, E), W_ip,
      b_ip.reshape(1, DP), noise)

    logits = pl.pallas_call(
        _expert_kernel,
        grid=(E,),
        in_specs=[
            pl.BlockSpec((B, DV), lambda e: (0, 0)),
            pl.BlockSpec((B, DP), lambda e: (0, 0)),
            pl.BlockSpec((1, B, 1), lambda e: (e, 0, 0)),
            pl.BlockSpec((1, DV, DFF), lambda e: (e, 0, 0)),
            pl.BlockSpec((1, DP, DFF), lambda e: (e, 0, 0)),
            pl.BlockSpec((1, 1, DFF), lambda e: (e, 0, 0)),
            pl.BlockSpec((1, DFF, NC), lambda e: (e, 0, 0)),
            pl.BlockSpec((1, 1, NC), lambda e: (e, 0, 0)),
        ],
        out_specs=pl.BlockSpec((B, NC), lambda e: (0, 0)),
        out_shape=jax.ShapeDtypeStruct((B, NC), jnp.float32),
    )(vision_input, tproj, mask.T.reshape(E, B, 1), W1v, W1t,
      b1.reshape(E, 1, DFF), W2, b2.reshape(E, 1, NC))

    return (logits, il.reshape(()), ent.reshape(()))
